# baseline (speedup 1.0000x reference)
import functools

import jax
import jax.numpy as jnp
from jax import lax
from jax.experimental import pallas as pl
from jax.experimental.pallas import tpu as pltpu
from jax.experimental.pallas import tpu_sc as plsc

N_NODES = 10000
D_FULL = 256
D_HALF = 128
N_EDGES = 160000
N_CLS = 40
N_TILES = 16
CH = 128
PAD_E = 1280 * CH
CPT = PAD_E // (N_TILES * CH)
SLAB = CPT // 2
STRIPE = 1000
ACC_ROWS = N_NODES + 16
ZSTRIPE = ACC_ROWS // N_TILES
DEG_N = 10240
DEG_CH = 128
DEG_J = PAD_E // (2 * N_TILES * DEG_CH)

_f32 = jnp.float32


def _deg_body(dstD, ones_h, zeros1, o0, o1, o2,
              kidx, ones_v, dsp0, dsp1, dsp2, dsem):
    c = lax.axis_index("c")
    s = lax.axis_index("s")
    pltpu.sync_copy(ones_h, ones_v)

    @pl.when(s < 8)
    def _():
        for dsp in (dsp0, dsp1, dsp2):
            pltpu.sync_copy(zeros1, dsp.at[pl.ds(s * 1280, 1280)])

    plsc.subcore_barrier()

    for k, dsp in enumerate((dsp0, dsp1, dsp2)):
        pltpu.sync_copy(dstD.at[k, c, s], kidx)

        def _fire(j, _, dsp=dsp):
            pltpu.async_copy(ones_v, dsp.at[kidx.at[j]], dsem, add=True)
            return 0

        lax.fori_loop(0, DEG_J, _fire, 0)

        def _drain(j, _, dsp=dsp):
            pltpu.make_async_copy(ones_v, dsp.at[kidx.at[0]], dsem).wait()
            return 0

        lax.fori_loop(0, DEG_J, _drain, 0)

    plsc.subcore_barrier()

    @pl.when(s == 0)
    def _():
        for dsp, o in ((dsp0, o0), (dsp1, o1), (dsp2, o2)):
            pltpu.sync_copy(dsp, o.at[pl.ds(c * DEG_N, DEG_N)])


_deg_call = pl.kernel(
    _deg_body,
    out_type=[jax.ShapeDtypeStruct((2 * DEG_N,), _f32)] * 3,
    mesh=plsc.VectorSubcoreMesh(core_axis_name="c", subcore_axis_name="s"),
    scratch_types=[
        pltpu.VMEM((DEG_J, DEG_CH), jnp.int32),
        pltpu.VMEM((DEG_CH,), _f32),
        pltpu.VMEM_SHARED((DEG_N,), _f32),
        pltpu.VMEM_SHARED((DEG_N,), _f32),
        pltpu.VMEM_SHARED((DEG_N,), _f32),
        pltpu.SemaphoreType.DMA,
    ],
)


def _edge_body(hs_flat, srcRC, dstR, zeros2d, out,
               sidx, didx, rows, accsp, gs0, gs1, ss0, ss1):
    c = lax.axis_index("c")
    s = lax.axis_index("s")
    gsems = (gs0, gs1)
    ssems = (ss0, ss1)

    pltpu.sync_copy(zeros2d, accsp.at[pl.ds(s * ZSTRIPE, ZSTRIPE)])
    plsc.subcore_barrier()

    def g_start(ci, b):
        pltpu.async_copy(hs_flat.at[sidx.at[ci]], rows.at[b], gsems[b])

    def g_wait(ci, b):
        pltpu.make_async_copy(hs_flat.at[sidx.at[ci]], rows.at[b],
                              gsems[b]).wait()

    def s_start(ci, b):
        pltpu.async_copy(rows.at[b], accsp.at[didx.at[ci]], ssems[b],
                         add=True)

    def s_wait(ci, b):
        pltpu.make_async_copy(rows.at[b], accsp.at[didx.at[ci]],
                              ssems[b]).wait()

    for st in range(2):
        row0 = s * CPT + st * SLAB
        pltpu.sync_copy(srcRC.at[c, pl.ds(row0, SLAB)], sidx)
        pltpu.sync_copy(dstR.at[pl.ds(row0, SLAB)], didx)
        g_start(0, 0)
        g_start(1, 1)

        def outer(i, _):
            for b in range(2):
                ci = 2 * i + b
                g_wait(ci, b)
                s_start(ci, b)

                @pl.when(ci + 2 < SLAB)
                def _():
                    s_wait(ci, b)
                    g_start(ci + 2, b)
            return 0

        lax.fori_loop(0, SLAB // 2, outer, 0)
        s_wait(SLAB - 2, 0)
        s_wait(SLAB - 1, 1)
    plsc.subcore_barrier()

    @pl.when(s < 10)
    def _():
        pltpu.sync_copy(accsp.at[pl.ds(s * STRIPE, STRIPE)],
                        out.at[pl.ds(c * N_NODES + s * STRIPE, STRIPE)])


_edge_call = pl.kernel(
    _edge_body,
    out_type=jax.ShapeDtypeStruct((2 * N_NODES, D_HALF), _f32),
    mesh=plsc.VectorSubcoreMesh(core_axis_name="c", subcore_axis_name="s"),
    scratch_types=[
        pltpu.VMEM((SLAB, CH), jnp.int32),
        pltpu.VMEM((SLAB, CH), jnp.int32),
        pltpu.VMEM((2, CH, D_HALF), _f32),
        pltpu.VMEM_SHARED((ACC_ROWS, D_HALF), _f32),
        pltpu.SemaphoreType.DMA,
        pltpu.SemaphoreType.DMA,
        pltpu.SemaphoreType.DMA,
        pltpu.SemaphoreType.DMA,
    ],
)


_R = 400
_GRID = (N_NODES // _R,)


_bf16 = jnp.bfloat16


def _tcA_body(x_ref, w_ref, dega_ref, degb_ref, hs_ref):
    h = jnp.dot(x_ref[...].astype(_bf16), w_ref[...].astype(_bf16),
                preferred_element_type=_f32)
    deg = dega_ref[...] + degb_ref[...] + 1.0
    dis = lax.rsqrt(jnp.maximum(deg, 1e-12))
    hs = h * dis
    hs_ref[0, :, :] = hs[:, :D_HALF]
    hs_ref[1, :, :] = hs[:, D_HALF:]


_tcA = pl.pallas_call(
    _tcA_body,
    grid=_GRID,
    in_specs=[
        pl.BlockSpec((_R, D_FULL), lambda i: (i, 0)),
        pl.BlockSpec((D_FULL, D_FULL), lambda i: (0, 0)),
        pl.BlockSpec((_R, 1), lambda i: (i, 0)),
        pl.BlockSpec((_R, 1), lambda i: (i, 0)),
    ],
    out_specs=pl.BlockSpec((2, _R, D_HALF), lambda i: (0, i, 0)),
    out_shape=jax.ShapeDtypeStruct((2, N_NODES, D_HALF), _f32),
)


def _tcB_body(a_ref, h_ref, dega_ref, degb_ref, b1_ref, w2_ref, o_ref):
    deg = dega_ref[...] + degb_ref[...] + 1.0
    dis = lax.rsqrt(jnp.maximum(deg, 1e-12))
    agg = jnp.concatenate([a_ref[0] + h_ref[0], a_ref[1] + h_ref[1]], axis=1)
    z = jnp.maximum(agg * dis + b1_ref[...], 0.0)
    h2 = jnp.dot(z.astype(_bf16), w2_ref[...].astype(_bf16),
                 preferred_element_type=_f32)
    hs2 = h2 * dis
    o_ref[0, :, :] = hs2[:, :D_HALF]
    o_ref[1, :, :] = hs2[:, D_HALF:]


_tcB = pl.pallas_call(
    _tcB_body,
    grid=_GRID,
    in_specs=[
        pl.BlockSpec((2, _R, D_HALF), lambda i: (0, i, 0)),
        pl.BlockSpec((2, _R, D_HALF), lambda i: (0, i, 0)),
        pl.BlockSpec((_R, 1), lambda i: (i, 0)),
        pl.BlockSpec((_R, 1), lambda i: (i, 0)),
        pl.BlockSpec((1, D_FULL), lambda i: (0, 0)),
        pl.BlockSpec((D_FULL, D_FULL), lambda i: (0, 0)),
    ],
    out_specs=pl.BlockSpec((2, _R, D_HALF), lambda i: (0, i, 0)),
    out_shape=jax.ShapeDtypeStruct((2, N_NODES, D_HALF), _f32),
)


def _tcC_body(*refs):
    b2s_ref, wf_ref, bf_ref, wc_ref, bc_ref, out_ref = refs[12:]
    y = None
    for k in range(3):
        a, h, dega, degb = refs[4 * k:4 * k + 4]
        deg = dega[...] + degb[...] + 1.0
        dis = lax.rsqrt(jnp.maximum(deg, 1e-12))
        g = jnp.concatenate([a[0] + h[0], a[1] + h[1]], axis=1)
        g = g * dis + b2s_ref[k:k + 1, :]
        t = jnp.dot(g.astype(_bf16), wf_ref[k].astype(_bf16),
                    preferred_element_type=_f32)
        y = t if y is None else y + t
    y = jnp.maximum(y + bf_ref[...], 0.0)
    out_ref[...] = jnp.dot(y.astype(_bf16), wc_ref[...].astype(_bf16),
                           preferred_element_type=_f32) + bc_ref[...]


_tcC = pl.pallas_call(
    _tcC_body,
    grid=_GRID,
    in_specs=(
        [pl.BlockSpec((2, _R, D_HALF), lambda i: (0, i, 0)),
         pl.BlockSpec((2, _R, D_HALF), lambda i: (0, i, 0)),
         pl.BlockSpec((_R, 1), lambda i: (i, 0)),
         pl.BlockSpec((_R, 1), lambda i: (i, 0))] * 3
        + [pl.BlockSpec((3, D_FULL), lambda i: (0, 0)),
           pl.BlockSpec((3, D_FULL, D_FULL), lambda i: (0, 0, 0)),
           pl.BlockSpec((1, D_FULL), lambda i: (0, 0)),
           pl.BlockSpec((D_FULL, N_CLS), lambda i: (0, 0)),
           pl.BlockSpec((1, N_CLS), lambda i: (0, 0))]
    ),
    out_specs=pl.BlockSpec((_R, N_CLS), lambda i: (i, 0)),
    out_shape=jax.ShapeDtypeStruct((N_NODES, N_CLS), _f32),
)


def _impl(x, edge_index_0, edge_index_1, edge_index_2,
          W1_0, b1_0, W1_1, b1_1, W1_2, b1_2,
          W2_0, b2_0, W2_1, b2_1, W2_2, b2_2,
          Wf, bf, Wc, bc, use_sc_deg=True, use_sc_edge=True, use_tc=True):
    eis = (edge_index_0, edge_index_1, edge_index_2)
    srcs = [e[0].astype(jnp.int32) for e in eis]
    dsts = [e[1].astype(jnp.int32) for e in eis]
    npad = PAD_E - N_EDGES
    pad_src = (jnp.arange(npad, dtype=jnp.int32) * 37) % N_NODES
    pad_dst = N_NODES + (jnp.arange(npad, dtype=jnp.int32) % 16)
    srcs_p = [jnp.concatenate([s, pad_src]) for s in srcs]
    dsts_p = [jnp.concatenate([d, pad_dst]) for d in dsts]
    ones_h = jnp.ones((DEG_CH,), _f32)
    zeros1 = jnp.zeros((1280,), _f32)
    zeros2d = jnp.zeros((ZSTRIPE, D_HALF), _f32)
    srcRC = [jnp.stack([s, s + N_NODES]).reshape(2, N_TILES * CPT, CH)
             for s in srcs_p]
    dstR = [d.reshape(N_TILES * CPT, CH) for d in dsts_p]
    dstD = jnp.stack(dsts_p).reshape(3, 2, N_TILES, DEG_J, DEG_CH)

    if use_sc_deg:
        degs = _deg_call(dstD, ones_h, zeros1)
        dega = [d[:N_NODES, None] for d in degs]
        degb = [d[DEG_N:DEG_N + N_NODES, None] for d in degs]
    else:
        dega = [jnp.zeros((N_NODES,), _f32).at[d].add(1.0)[:, None]
                for d in dsts]
        degb = [jnp.zeros((N_NODES, 1), _f32)] * 3

    W1s = (W1_0, W1_1, W1_2)
    b1s = (b1_0, b1_1, b1_2)
    W2s = (W2_0, W2_1, W2_2)

    def edge(hs, k):
        if use_sc_edge:
            acc = _edge_call(hs.reshape(2 * N_NODES, D_HALF),
                             srcRC[k], dstR[k], zeros2d)
            return acc.reshape(2, N_NODES, D_HALF)
        hs_full = jnp.concatenate([hs[0], hs[1]], axis=1)
        acc = jnp.zeros((N_NODES, D_FULL), _f32).at[dsts[k]].add(
            hs_full[srcs[k]])
        return jnp.stack([acc[:, :D_HALF], acc[:, D_HALF:]])

    acc2 = []
    for k in range(3):
        dis = lax.rsqrt(jnp.maximum(dega[k] + degb[k] + 1.0, 1e-12))
        if use_tc:
            hs = _tcA(x, W1s[k], dega[k], degb[k])
        else:
            h1 = (x @ W1s[k]) * dis
            hs = jnp.stack([h1[:, :D_HALF], h1[:, D_HALF:]])
        a = edge(hs, k)
        if use_tc:
            hs2 = _tcB(a, hs, dega[k], degb[k],
                       b1s[k].reshape(1, D_FULL), W2s[k])
        else:
            agg = jnp.concatenate([a[0] + hs[0], a[1] + hs[1]], axis=1)
            z = jnp.maximum(agg * dis + b1s[k][None, :], 0.0)
            h2 = (z @ W2s[k]) * dis
            hs2 = jnp.stack([h2[:, :D_HALF], h2[:, D_HALF:]])
        a2 = edge(hs2, k)
        acc2.append((a2, hs2))

    if use_tc:
        b2s = jnp.stack([b2_0, b2_1, b2_2])
        wf3 = Wf.reshape(3, D_FULL, D_FULL)
        args = []
        for k in range(3):
            a2, hs2 = acc2[k]
            args += [a2, hs2, dega[k], degb[k]]
        args += [b2s, wf3, bf.reshape(1, D_FULL), Wc, bc.reshape(1, N_CLS)]
        return _tcC(*args)
    outs = []
    b2ss = (b2_0, b2_1, b2_2)
    for k in range(3):
        a2, hs2 = acc2[k]
        dis = lax.rsqrt(jnp.maximum(dega[k] + degb[k] + 1.0, 1e-12))
        g = jnp.concatenate([a2[0] + hs2[0], a2[1] + hs2[1]], axis=1)
        outs.append(g * dis + b2ss[k][None, :])
    h = jnp.concatenate(outs, -1)
    h = jnp.maximum(h @ Wf + bf, 0.0)
    return h @ Wc + bc


def kernel(x, edge_index_0, edge_index_1, edge_index_2,
           W1_0, b1_0, W1_1, b1_1, W1_2, b1_2,
           W2_0, b2_0, W2_1, b2_1, W2_2, b2_2,
           Wf, bf, Wc, bc):
    return _impl(x, edge_index_0, edge_index_1, edge_index_2,
                 W1_0, b1_0, W1_1, b1_1, W1_2, b1_2,
                 W2_0, b2_0, W2_1, b2_1, W2_2, b2_2,
                 Wf, bf, Wc, bc,
                 use_sc_deg=True, use_sc_edge=True, use_tc=True)


# speedup vs baseline: 1.0067x; 1.0067x over previous
import functools

import jax
import jax.numpy as jnp
from jax import lax
from jax.experimental import pallas as pl
from jax.experimental.pallas import tpu as pltpu
from jax.experimental.pallas import tpu_sc as plsc

N_NODES = 10000
D_FULL = 256
D_HALF = 128
N_EDGES = 160000
N_CLS = 40
N_TILES = 16
CH = 128
PAD_E = 1280 * CH
CPT = PAD_E // (N_TILES * CH)
SLAB = CPT // 2
SSLAB = 16
NST = CPT // SSLAB
STRIPE = 1000
ACC_ROWS = N_NODES + 16
ZSTRIPE = ACC_ROWS // N_TILES
DEG_N = 10240
DEG_CH = 128
DEG_J = PAD_E // (2 * N_TILES * DEG_CH)

_f32 = jnp.float32


def _deg_body(dstD, ones_h, zeros1, o0, o1, o2,
              kidx, ones_v, dsp0, dsp1, dsp2, dsem):
    c = lax.axis_index("c")
    s = lax.axis_index("s")
    pltpu.sync_copy(ones_h, ones_v)

    @pl.when(s < 8)
    def _():
        for dsp in (dsp0, dsp1, dsp2):
            pltpu.sync_copy(zeros1, dsp.at[pl.ds(s * 1280, 1280)])

    plsc.subcore_barrier()

    for k, dsp in enumerate((dsp0, dsp1, dsp2)):
        pltpu.sync_copy(dstD.at[k, c, s], kidx)

        def _fire(j, _, dsp=dsp):
            pltpu.async_copy(ones_v, dsp.at[kidx.at[j]], dsem, add=True)
            return 0

        lax.fori_loop(0, DEG_J, _fire, 0)

        def _drain(j, _, dsp=dsp):
            pltpu.make_async_copy(ones_v, dsp.at[kidx.at[0]], dsem).wait()
            return 0

        lax.fori_loop(0, DEG_J, _drain, 0)

    plsc.subcore_barrier()

    @pl.when(s == 0)
    def _():
        for dsp, o in ((dsp0, o0), (dsp1, o1), (dsp2, o2)):
            pltpu.sync_copy(dsp, o.at[pl.ds(c * DEG_N, DEG_N)])


_deg_call = pl.kernel(
    _deg_body,
    out_type=[jax.ShapeDtypeStruct((2 * DEG_N,), _f32)] * 3,
    mesh=plsc.VectorSubcoreMesh(core_axis_name="c", subcore_axis_name="s"),
    scratch_types=[
        pltpu.VMEM((DEG_J, DEG_CH), jnp.int32),
        pltpu.VMEM((DEG_CH,), _f32),
        pltpu.VMEM_SHARED((DEG_N,), _f32),
        pltpu.VMEM_SHARED((DEG_N,), _f32),
        pltpu.VMEM_SHARED((DEG_N,), _f32),
        pltpu.SemaphoreType.DMA,
    ],
)


def _edge_body(hs_flat, srcRC, dstR, zeros2d, out,
               sidx, didx, rows, accsp, gs0, gs1, ss0, ss1, is0, is1):
    c = lax.axis_index("c")
    s = lax.axis_index("s")
    gsems = (gs0, gs1)
    ssems = (ss0, ss1)
    isems = (is0, is1)

    def i_load(st, p):
        row0 = s * CPT + st * SSLAB
        pltpu.async_copy(srcRC.at[c, pl.ds(row0, SSLAB)], sidx.at[p],
                         isems[p])
        pltpu.async_copy(dstR.at[pl.ds(row0, SSLAB)], didx.at[p], isems[p])

    def i_wait(st, p):
        row0 = s * CPT + st * SSLAB
        pltpu.make_async_copy(srcRC.at[c, pl.ds(row0, SSLAB)], sidx.at[p],
                              isems[p]).wait()
        pltpu.make_async_copy(dstR.at[pl.ds(row0, SSLAB)], didx.at[p],
                              isems[p]).wait()

    def g_start(p, l, b):
        pltpu.async_copy(hs_flat.at[sidx.at[p, l]], rows.at[b], gsems[b])

    def g_wait(p, l, b):
        pltpu.make_async_copy(hs_flat.at[sidx.at[p, l]], rows.at[b],
                              gsems[b]).wait()

    def s_start(p, l, b):
        pltpu.async_copy(rows.at[b], accsp.at[didx.at[p, l]], ssems[b],
                         add=True)

    def s_wait(p, l, b):
        pltpu.make_async_copy(rows.at[b], accsp.at[didx.at[p, l]],
                              ssems[b]).wait()

    i_load(0, 0)
    i_load(1, 1)
    pltpu.sync_copy(zeros2d, accsp.at[pl.ds(s * ZSTRIPE, ZSTRIPE)])
    plsc.subcore_barrier()
    i_wait(0, 0)
    g_start(0, 0, 0)
    g_start(0, 1, 1)

    for st in range(NST):
        p = st % 2

        def outer(i, _, p=p):
            for b in range(2):
                l = 2 * i + b
                g_wait(p, l, b)
                s_start(p, l, b)
                s_wait(p, l, b)
                g_start(p, l + 2, b)
            return 0

        lax.fori_loop(0, SSLAB // 2 - 1, outer, 0)
        if st + 1 < NST:
            i_wait(st + 1, (st + 1) % 2)
        for b in range(2):
            l = SSLAB - 2 + b
            g_wait(p, l, b)
            s_start(p, l, b)
            s_wait(p, l, b)
            if st + 1 < NST:
                g_start((st + 1) % 2, b, b)
        if st + 2 < NST:
            i_load(st + 2, p)
    plsc.subcore_barrier()

    @pl.when(s < 10)
    def _():
        pltpu.sync_copy(accsp.at[pl.ds(s * STRIPE, STRIPE)],
                        out.at[pl.ds(c * N_NODES + s * STRIPE, STRIPE)])


_edge_call = pl.kernel(
    _edge_body,
    out_type=jax.ShapeDtypeStruct((2 * N_NODES, D_HALF), _f32),
    mesh=plsc.VectorSubcoreMesh(core_axis_name="c", subcore_axis_name="s"),
    scratch_types=[
        pltpu.VMEM((2, SSLAB, CH), jnp.int32),
        pltpu.VMEM((2, SSLAB, CH), jnp.int32),
        pltpu.VMEM((2, CH, D_HALF), _f32),
        pltpu.VMEM_SHARED((ACC_ROWS, D_HALF), _f32),
        pltpu.SemaphoreType.DMA,
        pltpu.SemaphoreType.DMA,
        pltpu.SemaphoreType.DMA,
        pltpu.SemaphoreType.DMA,
        pltpu.SemaphoreType.DMA,
        pltpu.SemaphoreType.DMA,
    ],
)


_R = 400
_GRID = (N_NODES // _R,)


def _tcA_body(x_ref, w_ref, dega_ref, degb_ref, hs_ref):
    h = jnp.dot(x_ref[...], w_ref[...], preferred_element_type=_f32)
    deg = dega_ref[...] + degb_ref[...] + 1.0
    dis = lax.rsqrt(jnp.maximum(deg, 1e-12))
    hs = h * dis
    hs_ref[0, :, :] = hs[:, :D_HALF]
    hs_ref[1, :, :] = hs[:, D_HALF:]


_tcA = pl.pallas_call(
    _tcA_body,
    grid=_GRID,
    in_specs=[
        pl.BlockSpec((_R, D_FULL), lambda i: (i, 0)),
        pl.BlockSpec((D_FULL, D_FULL), lambda i: (0, 0)),
        pl.BlockSpec((_R, 1), lambda i: (i, 0)),
        pl.BlockSpec((_R, 1), lambda i: (i, 0)),
    ],
    out_specs=pl.BlockSpec((2, _R, D_HALF), lambda i: (0, i, 0)),
    out_shape=jax.ShapeDtypeStruct((2, N_NODES, D_HALF), _f32),
)


def _tcB_body(a_ref, h_ref, dega_ref, degb_ref, b1_ref, w2_ref, o_ref):
    deg = dega_ref[...] + degb_ref[...] + 1.0
    dis = lax.rsqrt(jnp.maximum(deg, 1e-12))
    agg = jnp.concatenate([a_ref[0] + h_ref[0], a_ref[1] + h_ref[1]], axis=1)
    z = jnp.maximum(agg * dis + b1_ref[...], 0.0)
    h2 = jnp.dot(z, w2_ref[...], preferred_element_type=_f32)
    hs2 = h2 * dis
    o_ref[0, :, :] = hs2[:, :D_HALF]
    o_ref[1, :, :] = hs2[:, D_HALF:]


_tcB = pl.pallas_call(
    _tcB_body,
    grid=_GRID,
    in_specs=[
        pl.BlockSpec((2, _R, D_HALF), lambda i: (0, i, 0)),
        pl.BlockSpec((2, _R, D_HALF), lambda i: (0, i, 0)),
        pl.BlockSpec((_R, 1), lambda i: (i, 0)),
        pl.BlockSpec((_R, 1), lambda i: (i, 0)),
        pl.BlockSpec((1, D_FULL), lambda i: (0, 0)),
        pl.BlockSpec((D_FULL, D_FULL), lambda i: (0, 0)),
    ],
    out_specs=pl.BlockSpec((2, _R, D_HALF), lambda i: (0, i, 0)),
    out_shape=jax.ShapeDtypeStruct((2, N_NODES, D_HALF), _f32),
)


def _tcC_body(*refs):
    b2s_ref, wf_ref, bf_ref, wc_ref, bc_ref, out_ref = refs[12:]
    y = None
    for k in range(3):
        a, h, dega, degb = refs[4 * k:4 * k + 4]
        deg = dega[...] + degb[...] + 1.0
        dis = lax.rsqrt(jnp.maximum(deg, 1e-12))
        g = jnp.concatenate([a[0] + h[0], a[1] + h[1]], axis=1)
        g = g * dis + b2s_ref[k:k + 1, :]
        t = jnp.dot(g, wf_ref[k], preferred_element_type=_f32)
        y = t if y is None else y + t
    y = jnp.maximum(y + bf_ref[...], 0.0)
    out_ref[...] = jnp.dot(y, wc_ref[...], preferred_element_type=_f32) \
        + bc_ref[...]


_tcC = pl.pallas_call(
    _tcC_body,
    grid=_GRID,
    in_specs=(
        [pl.BlockSpec((2, _R, D_HALF), lambda i: (0, i, 0)),
         pl.BlockSpec((2, _R, D_HALF), lambda i: (0, i, 0)),
         pl.BlockSpec((_R, 1), lambda i: (i, 0)),
         pl.BlockSpec((_R, 1), lambda i: (i, 0))] * 3
        + [pl.BlockSpec((3, D_FULL), lambda i: (0, 0)),
           pl.BlockSpec((3, D_FULL, D_FULL), lambda i: (0, 0, 0)),
           pl.BlockSpec((1, D_FULL), lambda i: (0, 0)),
           pl.BlockSpec((D_FULL, N_CLS), lambda i: (0, 0)),
           pl.BlockSpec((1, N_CLS), lambda i: (0, 0))]
    ),
    out_specs=pl.BlockSpec((_R, N_CLS), lambda i: (i, 0)),
    out_shape=jax.ShapeDtypeStruct((N_NODES, N_CLS), _f32),
)


def _impl(x, edge_index_0, edge_index_1, edge_index_2,
          W1_0, b1_0, W1_1, b1_1, W1_2, b1_2,
          W2_0, b2_0, W2_1, b2_1, W2_2, b2_2,
          Wf, bf, Wc, bc, use_sc_deg=True, use_sc_edge=True, use_tc=True):
    eis = (edge_index_0, edge_index_1, edge_index_2)
    srcs = [e[0].astype(jnp.int32) for e in eis]
    dsts = [e[1].astype(jnp.int32) for e in eis]
    npad = PAD_E - N_EDGES
    pad_src = (jnp.arange(npad, dtype=jnp.int32) * 37) % N_NODES
    pad_dst = N_NODES + (jnp.arange(npad, dtype=jnp.int32) % 16)
    srcs_p = [jnp.concatenate([s, pad_src]) for s in srcs]
    dsts_p = [jnp.concatenate([d, pad_dst]) for d in dsts]
    ones_h = jnp.ones((DEG_CH,), _f32)
    zeros1 = jnp.zeros((1280,), _f32)
    zeros2d = jnp.zeros((ZSTRIPE, D_HALF), _f32)
    srcRC = [jnp.stack([s, s + N_NODES]).reshape(2, N_TILES * CPT, CH)
             for s in srcs_p]
    dstR = [d.reshape(N_TILES * CPT, CH) for d in dsts_p]
    dstD = jnp.stack(dsts_p).reshape(3, 2, N_TILES, DEG_J, DEG_CH)

    if use_sc_deg:
        degs = _deg_call(dstD, ones_h, zeros1)
        dega = [d[:N_NODES, None] for d in degs]
        degb = [d[DEG_N:DEG_N + N_NODES, None] for d in degs]
    else:
        dega = [jnp.zeros((N_NODES,), _f32).at[d].add(1.0)[:, None]
                for d in dsts]
        degb = [jnp.zeros((N_NODES, 1), _f32)] * 3

    W1s = (W1_0, W1_1, W1_2)
    b1s = (b1_0, b1_1, b1_2)
    W2s = (W2_0, W2_1, W2_2)

    def edge(hs, k):
        if use_sc_edge:
            acc = _edge_call(hs.reshape(2 * N_NODES, D_HALF),
                             srcRC[k], dstR[k], zeros2d)
            return acc.reshape(2, N_NODES, D_HALF)
        hs_full = jnp.concatenate([hs[0], hs[1]], axis=1)
        acc = jnp.zeros((N_NODES, D_FULL), _f32).at[dsts[k]].add(
            hs_full[srcs[k]])
        return jnp.stack([acc[:, :D_HALF], acc[:, D_HALF:]])

    acc2 = []
    for k in range(3):
        dis = lax.rsqrt(jnp.maximum(dega[k] + degb[k] + 1.0, 1e-12))
        if use_tc:
            hs = _tcA(x, W1s[k], dega[k], degb[k])
        else:
            h1 = (x @ W1s[k]) * dis
            hs = jnp.stack([h1[:, :D_HALF], h1[:, D_HALF:]])
        a = edge(hs, k)
        if use_tc:
            hs2 = _tcB(a, hs, dega[k], degb[k],
                       b1s[k].reshape(1, D_FULL), W2s[k])
        else:
            agg = jnp.concatenate([a[0] + hs[0], a[1] + hs[1]], axis=1)
            z = jnp.maximum(agg * dis + b1s[k][None, :], 0.0)
            h2 = (z @ W2s[k]) * dis
            hs2 = jnp.stack([h2[:, :D_HALF], h2[:, D_HALF:]])
        a2 = edge(hs2, k)
        acc2.append((a2, hs2))

    if use_tc:
        b2s = jnp.stack([b2_0, b2_1, b2_2])
        wf3 = Wf.reshape(3, D_FULL, D_FULL)
        args = []
        for k in range(3):
            a2, hs2 = acc2[k]
            args += [a2, hs2, dega[k], degb[k]]
        args += [b2s, wf3, bf.reshape(1, D_FULL), Wc, bc.reshape(1, N_CLS)]
        return _tcC(*args)
    outs = []
    b2ss = (b2_0, b2_1, b2_2)
    for k in range(3):
        a2, hs2 = acc2[k]
        dis = lax.rsqrt(jnp.maximum(dega[k] + degb[k] + 1.0, 1e-12))
        g = jnp.concatenate([a2[0] + hs2[0], a2[1] + hs2[1]], axis=1)
        outs.append(g * dis + b2ss[k][None, :])
    h = jnp.concatenate(outs, -1)
    h = jnp.maximum(h @ Wf + bf, 0.0)
    return h @ Wc + bc


def kernel(x, edge_index_0, edge_index_1, edge_index_2,
           W1_0, b1_0, W1_1, b1_1, W1_2, b1_2,
           W2_0, b2_0, W2_1, b2_1, W2_2, b2_2,
           Wf, bf, Wc, bc):
    return _impl(x, edge_index_0, edge_index_1, edge_index_2,
                 W1_0, b1_0, W1_1, b1_1, W1_2, b1_2,
                 W2_0, b2_0, W2_1, b2_1, W2_2, b2_2,
                 Wf, bf, Wc, bc,
                 use_sc_deg=True, use_sc_edge=True, use_tc=True)
